# Initial kernel scaffold; baseline (speedup 1.0000x reference)
import jax
import jax.numpy as jnp
from jax.experimental import pallas as pl


def kernel(X, Z, combo_w, w):
    raise NotImplementedError("write your pallas kernel here")


# speedup vs baseline: 52.5526x; 52.5526x over previous
import functools

import jax
import jax.numpy as jnp
import numpy as np
from jax import lax
from jax.experimental import pallas as pl
from jax.experimental.pallas import tpu as pltpu
from jax.experimental.pallas import tpu_sc as plsc

_N = 4096
_NCELL = 512
_M = 32
_NNB = 27
_ROW = 896
_REAL = 864
_BA = 256


def _static_tables():
    mesh_args = [np.arange(0.0, 8.0, 1.0, dtype=np.float32) for _ in range(3)]
    cells = np.stack(np.meshgrid(*mesh_args), axis=-1).reshape(-1, 3)
    d_cc = ((cells[:, None, :] - cells[None, :, :]) ** 2).sum(-1)
    nc = np.argsort(d_cc, axis=1, kind="stable")[:, :_NNB].astype(np.int32)
    p = np.arange(_REAL)
    src = np.zeros((_NCELL, _ROW), np.int32)
    src[:, :_REAL] = nc[:, p // _M] * _M + (p % _M)[None, :]
    return src.reshape(-1)


_SRC_PAD = _static_tables()


def _k0_body(x0_ref, x1_ref, x2_ref, out_ref):
    c = lax.broadcasted_iota(jnp.int32, (1, _NCELL), 1)
    cx = ((c // 8) % 8).astype(jnp.float32)
    cy = (c // 64).astype(jnp.float32)
    cz = (c % 8).astype(jnp.float32)
    d = ((x0_ref[:, :] - cx) ** 2 + (x1_ref[:, :] - cy) ** 2) \
        + (x2_ref[:, :] - cz) ** 2
    m = jnp.min(d, axis=1, keepdims=True)
    sel = jnp.min(jnp.where(d == m, c, 99999), axis=1, keepdims=True)
    out_ref[:, :] = sel


def _run_k0(xc0, xc1, xc2):
    bs = pl.BlockSpec((_BA, 1), lambda i: (i, 0))
    return pl.pallas_call(
        _k0_body,
        grid=(_N // _BA,),
        in_specs=[bs, bs, bs],
        out_specs=pl.BlockSpec((_BA, 1), lambda i: (i, 0)),
        out_shape=jax.ShapeDtypeStruct((_N, 1), jnp.int32),
    )(xc0, xc1, xc2)


def _k1_body(xr0_ref, xr1_ref, xr2_ref, out_ref):
    pid = pl.program_id(0)
    cidx = pid * 64 + lax.broadcasted_iota(jnp.int32, (64, 1), 0)
    cx = ((cidx // 8) % 8).astype(jnp.float32)
    cy = (cidx // 64).astype(jnp.float32)
    cz = (cidx % 8).astype(jnp.float32)
    d = ((xr0_ref[:, :] - cx) ** 2 + (xr1_ref[:, :] - cy) ** 2) \
        + (xr2_ref[:, :] - cz) ** 2
    lane = lax.broadcasted_iota(jnp.int32, (1, _N), 1)
    for k in range(_M):
        m = jnp.min(d, axis=1, keepdims=True)
        idx = jnp.min(jnp.where(d == m, lane, 999999), axis=1, keepdims=True)
        out_ref[:, k:k + 1] = idx
        d = jnp.where(lane == idx, jnp.inf, d)


def _run_k1(xr0, xr1, xr2):
    bs = pl.BlockSpec((1, _N), lambda i: (0, 0))
    return pl.pallas_call(
        _k1_body,
        grid=(_NCELL // 64,),
        in_specs=[bs, bs, bs],
        out_specs=pl.BlockSpec((64, _M), lambda i: (i, 0)),
        out_shape=jax.ShapeDtypeStruct((_NCELL, _M), jnp.int32),
    )(xr0, xr1, xr2)


_TOT = _NCELL * _ROW
_CH = 1792


def _sc_gather_body(src_hbm, ca_hbm, x0_hbm, x1_hbm, x2_hbm,
                    r_out, gx_out, gy_out, gz_out,
                    ca_v, x0_v, x1_v, x2_v,
                    src_v, r_v, gx_v, gy_v, gz_v):
    info = plsc.get_sparse_core_info()
    nw = info.num_cores * info.num_subcores
    per_w = _TOT // nw
    nsub = per_w // _CH
    wid = lax.axis_index("s") * info.num_cores + lax.axis_index("c")
    pltpu.sync_copy(ca_hbm, ca_v)
    pltpu.sync_copy(x0_hbm, x0_v)
    pltpu.sync_copy(x1_hbm, x1_v)
    pltpu.sync_copy(x2_hbm, x2_v)

    def subchunk(s, _):
        base = wid * per_w + s * _CH
        pltpu.sync_copy(src_hbm.at[pl.ds(base, _CH)], src_v)

        def body(i, _):
            sl = pl.ds(i * 16, 16)
            idx = src_v[sl]
            rr = plsc.load_gather(ca_v, [idx])
            r_v[sl] = rr
            gx_v[sl] = plsc.load_gather(x0_v, [rr])
            gy_v[sl] = plsc.load_gather(x1_v, [rr])
            gz_v[sl] = plsc.load_gather(x2_v, [rr])
            return 0

        lax.fori_loop(0, _CH // 16, body, 0)
        pltpu.sync_copy(r_v, r_out.at[pl.ds(base, _CH)])
        pltpu.sync_copy(gx_v, gx_out.at[pl.ds(base, _CH)])
        pltpu.sync_copy(gy_v, gy_out.at[pl.ds(base, _CH)])
        pltpu.sync_copy(gz_v, gz_out.at[pl.ds(base, _CH)])
        return 0

    lax.fori_loop(0, nsub, subchunk, 0)


def _run_k2(src, ca_flat, x0, x1, x2):
    f32 = jnp.float32
    k = functools.partial(
        pl.kernel,
        mesh=plsc.VectorSubcoreMesh(core_axis_name="c", subcore_axis_name="s"),
        compiler_params=pltpu.CompilerParams(needs_layout_passes=False),
        out_type=[
            jax.ShapeDtypeStruct((_TOT,), jnp.int32),
            jax.ShapeDtypeStruct((_TOT,), f32),
            jax.ShapeDtypeStruct((_TOT,), f32),
            jax.ShapeDtypeStruct((_TOT,), f32),
        ],
        scratch_types=[
            pltpu.VMEM((_NCELL * _M,), jnp.int32),
            pltpu.VMEM((_N,), f32),
            pltpu.VMEM((_N,), f32),
            pltpu.VMEM((_N,), f32),
            pltpu.VMEM((_CH,), jnp.int32),
            pltpu.VMEM((_CH,), jnp.int32),
            pltpu.VMEM((_CH,), f32),
            pltpu.VMEM((_CH,), f32),
            pltpu.VMEM((_CH,), f32),
        ],
    )(_sc_gather_body)
    return k(src, ca_flat, x0, x1, x2)


def _k3_body(r_ref, vm_ref):
    r = r_ref[:, :]
    lane32 = lax.broadcasted_iota(jnp.int32, (1, _M), 1)
    parts = [jnp.where(lane32 == 0, 0.0, 1.0)
             * jnp.ones((r.shape[0], _M), jnp.float32)]
    for b2 in range(1, _NNB):
        p = r[:, b2 * _M:(b2 + 1) * _M]
        q = r[:, : b2 * _M]
        dup = jnp.any(p[:, :, None] == q[:, None, :], axis=2)
        parts.append(jnp.where(dup, 0.0, 1.0))
    parts.append(jnp.zeros((r.shape[0], _ROW - _REAL), jnp.float32))
    vm_ref[:, :] = jnp.concatenate(parts, axis=1)


def _run_k3(r_rows):
    return pl.pallas_call(
        _k3_body,
        grid=(_NCELL // 8,),
        in_specs=[pl.BlockSpec((8, _ROW), lambda i: (i, 0))],
        out_specs=pl.BlockSpec((8, _ROW), lambda i: (i, 0)),
        out_shape=jax.ShapeDtypeStruct((_NCELL, _ROW), jnp.float32),
    )(r_rows)


def _k4_body(cids_ref, gx_ref, gy_ref, gz_ref, vm_ref,
             x0_ref, x1_ref, x2_ref, xl_ref, cw_ref, wv_ref,
             out_ref, gxs, gys, gzs, vms):
    pid = pl.program_id(0)

    def copy_row(a, _):
        cid = cids_ref[pid * _BA + a]
        gxs[pl.ds(a, 1), :] = gx_ref[pl.ds(cid, 1), :]
        gys[pl.ds(a, 1), :] = gy_ref[pl.ds(cid, 1), :]
        gzs[pl.ds(a, 1), :] = gz_ref[pl.ds(cid, 1), :]
        vms[pl.ds(a, 1), :] = vm_ref[pl.ds(cid, 1), :]
        return 0

    lax.fori_loop(0, _BA, copy_row, 0)

    xb, yb, zb = x0_ref[:, :], x1_ref[:, :], x2_ref[:, :]
    d = ((xb - gxs[:, :]) ** 2 + (yb - gys[:, :]) ** 2) + (zb - gzs[:, :]) ** 2
    vm = vms[:, :]
    dm = jnp.where(vm > 0.5, d, jnp.inf)

    c0, c1, c2 = cw_ref[0, 0], cw_ref[0, 1], cw_ref[0, 2]
    c3, c4 = cw_ref[0, 3], cw_ref[0, 4]
    w0 = wv_ref[0, 0]

    def f(dd):
        rep = jnp.where(dd < 0.0, dd ** 2, 0.0)
        hyd = jnp.where(dd < 0.5, 1.0, jnp.where(dd < 1.5, 1.5 - dd, 0.0))
        hb = jnp.where(dd < -0.7, 1.0,
                       jnp.where(dd < 0.0, (1.0 / 0.7) * (0.0 - dd), 0.0))
        g1 = jnp.exp(-(dd / 0.5) ** 2)
        g2 = jnp.exp(-((dd - 3.0) / 2.0) ** 2)
        inter = c0 * rep + c1 * hyd + c2 * hb + c3 * g1 + c4 * g2
        return jnp.where(dd < 8.0, inter, 0.0) / (1.0 + w0 * 1.0)

    nv = jnp.sum(vm, axis=1, keepdims=True)
    padc = jnp.maximum(0.0, 32.0 - nv)
    xl, yl, zl = xl_ref[0, 0], xl_ref[0, 1], xl_ref[0, 2]
    dl = ((xb - xl) ** 2 + (yb - yl) ** 2) + (zb - zl) ** 2
    total = jnp.sum(padc * f(dl))

    lane = lax.broadcasted_iota(jnp.int32, (1, _ROW), 1)
    acc = jnp.zeros((_BA, 1), jnp.float32)
    for _ in range(_M):
        m = jnp.min(dm, axis=1, keepdims=True)
        acc = acc + f(m)
        sel = jnp.min(jnp.where(dm == m, lane, 99999), axis=1, keepdims=True)
        dm = jnp.where(lane == sel, jnp.inf, dm)
    total = total + jnp.sum(acc)

    @pl.when(pid == 0)
    def _():
        out_ref[0, 0] = 0.0

    out_ref[0, 0] += total


def _run_k4(cids, gx, gy, gz, vm, xc0, xc1, xc2, xlast, cw, wv):
    full = pl.BlockSpec((_NCELL, _ROW), lambda i, c: (0, 0))
    col = pl.BlockSpec((_BA, 1), lambda i, c: (i, 0))
    smem = pl.BlockSpec(memory_space=pltpu.SMEM)
    grid_spec = pltpu.PrefetchScalarGridSpec(
        num_scalar_prefetch=1,
        grid=(_N // _BA,),
        in_specs=[full, full, full, full, col, col, col, smem, smem, smem],
        out_specs=pl.BlockSpec(memory_space=pltpu.SMEM),
        scratch_shapes=[pltpu.VMEM((_BA, _ROW), jnp.float32)] * 4,
    )
    return pl.pallas_call(
        _k4_body,
        grid_spec=grid_spec,
        out_shape=jax.ShapeDtypeStruct((1, 1), jnp.float32),
    )(cids, gx, gy, gz, vm, xc0, xc1, xc2, xlast, cw, wv)


def kernel(X, Z, combo_w, w):
    X = jnp.asarray(X, jnp.float32)
    x0, x1, x2 = X[:, 0], X[:, 1], X[:, 2]
    xc0, xc1, xc2 = x0[:, None], x1[:, None], x2[:, None]
    xr0, xr1, xr2 = x0[None, :], x1[None, :], x2[None, :]

    cids = _run_k0(xc0, xc1, xc2).reshape(_N)
    closest = _run_k1(xr0, xr1, xr2)
    src = jnp.asarray(_SRC_PAD)
    r_flat, gxf, gyf, gzf = _run_k2(src, closest.reshape(-1), x0, x1, x2)
    vm = _run_k3(r_flat.reshape(_NCELL, _ROW))
    out = _run_k4(
        cids,
        gxf.reshape(_NCELL, _ROW), gyf.reshape(_NCELL, _ROW),
        gzf.reshape(_NCELL, _ROW), vm,
        xc0, xc1, xc2,
        X[_N - 1:_N, :],
        combo_w.reshape(1, 5).astype(jnp.float32),
        w.reshape(1, 1).astype(jnp.float32),
    )
    return out[0, 0]


# speedup vs baseline: 66.7123x; 1.2694x over previous
import functools

import jax
import jax.numpy as jnp
import numpy as np
from jax import lax
from jax.experimental import pallas as pl
from jax.experimental.pallas import tpu as pltpu
from jax.experimental.pallas import tpu_sc as plsc

_N = 4096
_NCELL = 512
_M = 32
_NNB = 27
_ROW = 896
_REAL = 864
_BA = 256


def _static_tables():
    mesh_args = [np.arange(0.0, 8.0, 1.0, dtype=np.float32) for _ in range(3)]
    cells = np.stack(np.meshgrid(*mesh_args), axis=-1).reshape(-1, 3)
    d_cc = ((cells[:, None, :] - cells[None, :, :]) ** 2).sum(-1)
    nc = np.argsort(d_cc, axis=1, kind="stable")[:, :_NNB].astype(np.int32)
    p = np.arange(_REAL)
    src = np.zeros((_NCELL, _ROW), np.int32)
    src[:, :_REAL] = nc[:, p // _M] * _M + (p % _M)[None, :]
    return src.reshape(-1)


_SRC_PAD = _static_tables()


def _k0_body(x0_ref, x1_ref, x2_ref, out_ref):
    c = lax.broadcasted_iota(jnp.int32, (1, _NCELL), 1)
    cx = ((c // 8) % 8).astype(jnp.float32)
    cy = (c // 64).astype(jnp.float32)
    cz = (c % 8).astype(jnp.float32)
    d = ((x0_ref[:, :] - cx) ** 2 + (x1_ref[:, :] - cy) ** 2) \
        + (x2_ref[:, :] - cz) ** 2
    m = jnp.min(d, axis=1, keepdims=True)
    sel = jnp.min(jnp.where(d == m, c, 99999), axis=1, keepdims=True)
    out_ref[:, :] = sel


def _run_k0(xc0, xc1, xc2):
    bs = pl.BlockSpec((_BA, 1), lambda i: (i, 0))
    return pl.pallas_call(
        _k0_body,
        grid=(_N // _BA,),
        in_specs=[bs, bs, bs],
        out_specs=pl.BlockSpec((_BA, 1), lambda i: (i, 0)),
        out_shape=jax.ShapeDtypeStruct((_N, 1), jnp.int32),
    )(xc0, xc1, xc2)


def _k1_body(xr0_ref, xr1_ref, xr2_ref, out_ref):
    pid = pl.program_id(0)
    cidx = pid * 64 + lax.broadcasted_iota(jnp.int32, (64, 1), 0)
    cx = ((cidx // 8) % 8).astype(jnp.float32)
    cy = (cidx // 64).astype(jnp.float32)
    cz = (cidx % 8).astype(jnp.float32)
    d = ((xr0_ref[:, :] - cx) ** 2 + (xr1_ref[:, :] - cy) ** 2) \
        + (xr2_ref[:, :] - cz) ** 2
    lane = lax.broadcasted_iota(jnp.int32, (1, _N), 1)
    for k in range(_M):
        m = jnp.min(d, axis=1, keepdims=True)
        idx = jnp.min(jnp.where(d == m, lane, 999999), axis=1, keepdims=True)
        out_ref[:, k:k + 1] = idx
        d = jnp.where(lane == idx, jnp.inf, d)


def _run_k1(xr0, xr1, xr2):
    bs = pl.BlockSpec((1, _N), lambda i: (0, 0))
    return pl.pallas_call(
        _k1_body,
        grid=(_NCELL // 64,),
        in_specs=[bs, bs, bs],
        out_specs=pl.BlockSpec((64, _M), lambda i: (i, 0)),
        out_shape=jax.ShapeDtypeStruct((_NCELL, _M), jnp.int32),
    )(xr0, xr1, xr2)


_TOT = _NCELL * _ROW
_CH = 1792


def _sc_gather_body(src_hbm, ca_hbm, x0_hbm, x1_hbm, x2_hbm,
                    r_out, gx_out, gy_out, gz_out,
                    ca_v, x0_v, x1_v, x2_v,
                    src_v, r_v, gx_v, gy_v, gz_v):
    info = plsc.get_sparse_core_info()
    nw = info.num_cores * info.num_subcores
    per_w = _TOT // nw
    nsub = per_w // _CH
    wid = lax.axis_index("s") * info.num_cores + lax.axis_index("c")
    pltpu.sync_copy(ca_hbm, ca_v)
    pltpu.sync_copy(x0_hbm, x0_v)
    pltpu.sync_copy(x1_hbm, x1_v)
    pltpu.sync_copy(x2_hbm, x2_v)

    def subchunk(s, _):
        base = wid * per_w + s * _CH
        pltpu.sync_copy(src_hbm.at[pl.ds(base, _CH)], src_v)

        def body(i, _):
            sl = pl.ds(i * 16, 16)
            idx = src_v[sl]
            rr = plsc.load_gather(ca_v, [idx])
            r_v[sl] = rr
            gx_v[sl] = plsc.load_gather(x0_v, [rr])
            gy_v[sl] = plsc.load_gather(x1_v, [rr])
            gz_v[sl] = plsc.load_gather(x2_v, [rr])
            return 0

        lax.fori_loop(0, _CH // 16, body, 0)
        pltpu.sync_copy(r_v, r_out.at[pl.ds(base, _CH)])
        pltpu.sync_copy(gx_v, gx_out.at[pl.ds(base, _CH)])
        pltpu.sync_copy(gy_v, gy_out.at[pl.ds(base, _CH)])
        pltpu.sync_copy(gz_v, gz_out.at[pl.ds(base, _CH)])
        return 0

    lax.fori_loop(0, nsub, subchunk, 0)


def _run_k2(src, ca_flat, x0, x1, x2):
    f32 = jnp.float32
    k = functools.partial(
        pl.kernel,
        mesh=plsc.VectorSubcoreMesh(core_axis_name="c", subcore_axis_name="s"),
        compiler_params=pltpu.CompilerParams(needs_layout_passes=False),
        out_type=[
            jax.ShapeDtypeStruct((_TOT,), jnp.int32),
            jax.ShapeDtypeStruct((_TOT,), f32),
            jax.ShapeDtypeStruct((_TOT,), f32),
            jax.ShapeDtypeStruct((_TOT,), f32),
        ],
        scratch_types=[
            pltpu.VMEM((_NCELL * _M,), jnp.int32),
            pltpu.VMEM((_N,), f32),
            pltpu.VMEM((_N,), f32),
            pltpu.VMEM((_N,), f32),
            pltpu.VMEM((_CH,), jnp.int32),
            pltpu.VMEM((_CH,), jnp.int32),
            pltpu.VMEM((_CH,), f32),
            pltpu.VMEM((_CH,), f32),
            pltpu.VMEM((_CH,), f32),
        ],
    )(_sc_gather_body)
    return k(src, ca_flat, x0, x1, x2)


def _k3_body(r_ref, gx_ref, gxo_ref):
    r = r_ref[:, :]
    lane32 = lax.broadcasted_iota(jnp.int32, (1, _M), 1)
    parts = [jnp.where(lane32 == 0, 1000.0, 0.0)
             + jnp.zeros((r.shape[0], _M), jnp.float32)]
    for b2 in range(1, _NNB):
        p = r[:, b2 * _M:(b2 + 1) * _M]
        q = r[:, : b2 * _M]
        dup = jnp.any(p[:, :, None] == q[:, None, :], axis=2)
        parts.append(jnp.where(dup, 1000.0, 0.0))
    parts.append(jnp.full((r.shape[0], _ROW - _REAL), 1000.0, jnp.float32))
    gxo_ref[:, :] = gx_ref[:, :] + jnp.concatenate(parts, axis=1)


def _run_k3(r_rows, gx_rows):
    bs = pl.BlockSpec((8, _ROW), lambda i: (i, 0))
    return pl.pallas_call(
        _k3_body,
        grid=(_NCELL // 8,),
        in_specs=[bs, bs],
        out_specs=bs,
        out_shape=jax.ShapeDtypeStruct((_NCELL, _ROW), jnp.float32),
    )(r_rows, gx_rows)


def _k4_body(cids_ref, gx_ref, gy_ref, gz_ref,
             x0_ref, x1_ref, x2_ref, xl_ref, cw_ref, wv_ref,
             out_ref, gxs, gys, gzs):
    pid = pl.program_id(0)

    def copy_row(a, _):
        cid = cids_ref[pid * _BA + a]
        gxs[pl.ds(a, 1), :] = gx_ref[pl.ds(cid, 1), :]
        gys[pl.ds(a, 1), :] = gy_ref[pl.ds(cid, 1), :]
        gzs[pl.ds(a, 1), :] = gz_ref[pl.ds(cid, 1), :]
        return 0

    lax.fori_loop(0, _BA, copy_row, 0)

    xb, yb, zb = x0_ref[:, :], x1_ref[:, :], x2_ref[:, :]
    d = ((xb - gxs[:, :]) ** 2 + (yb - gys[:, :]) ** 2) + (zb - gzs[:, :]) ** 2

    c0, c1, c2 = cw_ref[0, 0], cw_ref[0, 1], cw_ref[0, 2]
    c3, c4 = cw_ref[0, 3], cw_ref[0, 4]
    w0 = wv_ref[0, 0]

    def f(dd):
        rep = jnp.where(dd < 0.0, dd ** 2, 0.0)
        hyd = jnp.where(dd < 0.5, 1.0, jnp.where(dd < 1.5, 1.5 - dd, 0.0))
        hb = jnp.where(dd < -0.7, 1.0,
                       jnp.where(dd < 0.0, (1.0 / 0.7) * (0.0 - dd), 0.0))
        g1 = jnp.exp(-(dd / 0.5) ** 2)
        g2 = jnp.exp(-((dd - 3.0) / 2.0) ** 2)
        inter = c0 * rep + c1 * hyd + c2 * hb + c3 * g1 + c4 * g2
        return jnp.where(dd < 8.0, inter, 0.0) / (1.0 + w0 * 1.0)

    nv = jnp.sum(jnp.where(d < 192.0, 1.0, 0.0), axis=1, keepdims=True)
    padc = jnp.maximum(0.0, 32.0 - nv)
    xl, yl, zl = xl_ref[0, 0], xl_ref[0, 1], xl_ref[0, 2]
    dl = ((xb - xl) ** 2 + (yb - yl) ** 2) + (zb - zl) ** 2
    total = jnp.sum(padc * f(dl))

    keys = lax.bitcast_convert_type(d, jnp.int32)
    t = jnp.zeros((_BA, 1), jnp.int32)
    for b in range(30, -1, -1):
        cand = t | (1 << b)
        cnt = jnp.sum(jnp.where(keys < cand, 1.0, 0.0), axis=1, keepdims=True)
        t = jnp.where(cnt >= 32.0, t, cand)
    less = keys < t
    cntl = jnp.sum(jnp.where(less, 1.0, 0.0), axis=1, keepdims=True)
    t_f = lax.bitcast_convert_type(t, jnp.float32)
    total = total + jnp.sum(jnp.where(less, f(d), 0.0))
    total = total + jnp.sum((32.0 - cntl) * f(t_f))

    @pl.when(pid == 0)
    def _():
        out_ref[0, 0] = 0.0

    out_ref[0, 0] += total


def _run_k4(cids, gx, gy, gz, xc0, xc1, xc2, xlast, cw, wv):
    full = pl.BlockSpec((_NCELL, _ROW), lambda i, c: (0, 0))
    col = pl.BlockSpec((_BA, 1), lambda i, c: (i, 0))
    smem = pl.BlockSpec(memory_space=pltpu.SMEM)
    grid_spec = pltpu.PrefetchScalarGridSpec(
        num_scalar_prefetch=1,
        grid=(_N // _BA,),
        in_specs=[full, full, full, col, col, col, smem, smem, smem],
        out_specs=pl.BlockSpec(memory_space=pltpu.SMEM),
        scratch_shapes=[pltpu.VMEM((_BA, _ROW), jnp.float32)] * 3,
    )
    return pl.pallas_call(
        _k4_body,
        grid_spec=grid_spec,
        out_shape=jax.ShapeDtypeStruct((1, 1), jnp.float32),
    )(cids, gx, gy, gz, xc0, xc1, xc2, xlast, cw, wv)


def kernel(X, Z, combo_w, w):
    X = jnp.asarray(X, jnp.float32)
    x0, x1, x2 = X[:, 0], X[:, 1], X[:, 2]
    xc0, xc1, xc2 = x0[:, None], x1[:, None], x2[:, None]
    xr0, xr1, xr2 = x0[None, :], x1[None, :], x2[None, :]

    cids = _run_k0(xc0, xc1, xc2).reshape(_N)
    closest = _run_k1(xr0, xr1, xr2)
    src = jnp.asarray(_SRC_PAD)
    r_flat, gxf, gyf, gzf = _run_k2(src, closest.reshape(-1), x0, x1, x2)
    gxp = _run_k3(r_flat.reshape(_NCELL, _ROW), gxf.reshape(_NCELL, _ROW))
    out = _run_k4(
        cids,
        gxp, gyf.reshape(_NCELL, _ROW), gzf.reshape(_NCELL, _ROW),
        xc0, xc1, xc2,
        X[_N - 1:_N, :],
        combo_w.reshape(1, 5).astype(jnp.float32),
        w.reshape(1, 1).astype(jnp.float32),
    )
    return out[0, 0]


# speedup vs baseline: 115.9087x; 1.7374x over previous
import functools

import jax
import jax.numpy as jnp
import numpy as np
from jax import lax
from jax.experimental import pallas as pl
from jax.experimental.pallas import tpu as pltpu
from jax.experimental.pallas import tpu_sc as plsc

_N = 4096
_NCELL = 512
_M = 32
_NNB = 27
_ROW = 896
_REAL = 864
_BA = 256


def _static_tables():
    mesh_args = [np.arange(0.0, 8.0, 1.0, dtype=np.float32) for _ in range(3)]
    cells = np.stack(np.meshgrid(*mesh_args), axis=-1).reshape(-1, 3)
    d_cc = ((cells[:, None, :] - cells[None, :, :]) ** 2).sum(-1)
    nc = np.argsort(d_cc, axis=1, kind="stable")[:, :_NNB].astype(np.int32)
    p = np.arange(_REAL)
    src = np.zeros((_NCELL, _ROW), np.int32)
    src[:, :_REAL] = nc[:, p // _M] * _M + (p % _M)[None, :]
    return src.reshape(-1)


_SRC_PAD = _static_tables()


def _k0_body(x0_ref, x1_ref, x2_ref, out_ref):
    c = lax.broadcasted_iota(jnp.int32, (1, _NCELL), 1)
    cx = ((c // 8) % 8).astype(jnp.float32)
    cy = (c // 64).astype(jnp.float32)
    cz = (c % 8).astype(jnp.float32)
    d = ((x0_ref[:, :] - cx) ** 2 + (x1_ref[:, :] - cy) ** 2) \
        + (x2_ref[:, :] - cz) ** 2
    m = jnp.min(d, axis=1, keepdims=True)
    sel = jnp.min(jnp.where(d == m, c, 99999), axis=1, keepdims=True)
    out_ref[:, :] = sel


def _run_k0(xc0, xc1, xc2):
    bs = pl.BlockSpec((_BA, 1), lambda i: (i, 0))
    return pl.pallas_call(
        _k0_body,
        grid=(_N // _BA,),
        in_specs=[bs, bs, bs],
        out_specs=pl.BlockSpec((_BA, 1), lambda i: (i, 0)),
        out_shape=jax.ShapeDtypeStruct((_N, 1), jnp.int32),
    )(xc0, xc1, xc2)


def _k1_body(xr0_ref, xr1_ref, xr2_ref, out_ref):
    pid = pl.program_id(0)
    cidx = pid * 64 + lax.broadcasted_iota(jnp.int32, (64, 1), 0)
    cx = ((cidx // 8) % 8).astype(jnp.float32)
    cy = (cidx // 64).astype(jnp.float32)
    cz = (cidx % 8).astype(jnp.float32)
    d = ((xr0_ref[:, :] - cx) ** 2 + (xr1_ref[:, :] - cy) ** 2) \
        + (xr2_ref[:, :] - cz) ** 2
    lane = lax.broadcasted_iota(jnp.int32, (1, _N), 1)
    for k in range(_M):
        m = jnp.min(d, axis=1, keepdims=True)
        idx = jnp.min(jnp.where(d == m, lane, 999999), axis=1, keepdims=True)
        out_ref[:, k:k + 1] = idx
        d = jnp.where(lane == idx, jnp.inf, d)


def _run_k1(xr0, xr1, xr2):
    bs = pl.BlockSpec((1, _N), lambda i: (0, 0))
    return pl.pallas_call(
        _k1_body,
        grid=(_NCELL // 64,),
        in_specs=[bs, bs, bs],
        out_specs=pl.BlockSpec((64, _M), lambda i: (i, 0)),
        out_shape=jax.ShapeDtypeStruct((_NCELL, _M), jnp.int32),
    )(xr0, xr1, xr2)


_TOT = _NCELL * _ROW
_CH = 1792


def _sc_gather_body(src_hbm, ca_hbm, x0_hbm, x1_hbm, x2_hbm,
                    gx_out, gy_out, gz_out,
                    ca_v, x0_v, x1_v, x2_v,
                    src_v, gx_v, gy_v, gz_v, marks):
    info = plsc.get_sparse_core_info()
    nw = info.num_cores * info.num_subcores
    per_w = _TOT // nw
    nsub = per_w // _CH
    wid = lax.axis_index("s") * info.num_cores + lax.axis_index("c")
    pltpu.sync_copy(ca_hbm, ca_v)
    pltpu.sync_copy(x0_hbm, x0_v)
    pltpu.sync_copy(x1_hbm, x1_v)
    pltpu.sync_copy(x2_hbm, x2_v)

    lane = lax.broadcasted_iota(jnp.int32, (16,), 0)

    def initm(j, _):
        marks[pl.ds(j * 16, 16)] = jnp.full((16,), 16 * _NCELL, jnp.int32)
        return 0

    lax.fori_loop(0, _N // 16, initm, 0)

    def subchunk(s, _):
        base = wid * per_w + s * _CH
        pltpu.sync_copy(src_hbm.at[pl.ds(base, _CH)], src_v)

        def body(i, _):
            sl = pl.ds(i * 16, 16)
            pos = base + i * 16
            c = pos // _ROW
            pic = pos % _ROW
            idx = src_v[sl]
            rr = plsc.load_gather(ca_v, [idx])
            premask = jnp.logical_and(lane == 0, pic == 0)
            plsc.store_scatter(marks, [rr],
                               jnp.full((16,), 1, jnp.int32) * (c * 16),
                               mask=premask)
            prev = plsc.load_gather(marks, [rr])
            dupprev = lax.shift_right_logical(prev, 4) == c
            tok = c * 16 + lane
            plsc.store_scatter(marks, [rr], tok)
            back = plsc.load_gather(marks, [rr])
            invalid = jnp.logical_or(jnp.logical_or(back != tok, dupprev),
                                     pic >= _REAL)
            gxv = plsc.load_gather(x0_v, [rr])
            gx_v[sl] = jnp.where(invalid, gxv + 1000.0, gxv)
            gy_v[sl] = plsc.load_gather(x1_v, [rr])
            gz_v[sl] = plsc.load_gather(x2_v, [rr])
            return 0

        lax.fori_loop(0, _CH // 16, body, 0)
        pltpu.sync_copy(gx_v, gx_out.at[pl.ds(base, _CH)])
        pltpu.sync_copy(gy_v, gy_out.at[pl.ds(base, _CH)])
        pltpu.sync_copy(gz_v, gz_out.at[pl.ds(base, _CH)])
        return 0

    lax.fori_loop(0, nsub, subchunk, 0)


def _run_k2(src, ca_flat, x0, x1, x2):
    f32 = jnp.float32
    k = functools.partial(
        pl.kernel,
        mesh=plsc.VectorSubcoreMesh(core_axis_name="c", subcore_axis_name="s"),
        compiler_params=pltpu.CompilerParams(needs_layout_passes=False),
        out_type=[
            jax.ShapeDtypeStruct((_TOT,), f32),
            jax.ShapeDtypeStruct((_TOT,), f32),
            jax.ShapeDtypeStruct((_TOT,), f32),
        ],
        scratch_types=[
            pltpu.VMEM((_NCELL * _M,), jnp.int32),
            pltpu.VMEM((_N,), f32),
            pltpu.VMEM((_N,), f32),
            pltpu.VMEM((_N,), f32),
            pltpu.VMEM((_CH,), jnp.int32),
            pltpu.VMEM((_CH,), f32),
            pltpu.VMEM((_CH,), f32),
            pltpu.VMEM((_CH,), f32),
            pltpu.VMEM((_N,), jnp.int32),
        ],
    )(_sc_gather_body)
    return k(src, ca_flat, x0, x1, x2)


def _k4_body(cids_ref, gx_ref, gy_ref, gz_ref,
             x0_ref, x1_ref, x2_ref, xl_ref, cw_ref, wv_ref,
             out_ref, gxs, gys, gzs):
    pid = pl.program_id(0)

    def copy_row(a, _):
        cid = cids_ref[pid * _BA + a]
        gxs[pl.ds(a, 1), :] = gx_ref[pl.ds(cid, 1), :]
        gys[pl.ds(a, 1), :] = gy_ref[pl.ds(cid, 1), :]
        gzs[pl.ds(a, 1), :] = gz_ref[pl.ds(cid, 1), :]
        return 0

    lax.fori_loop(0, _BA, copy_row, 0)

    xb, yb, zb = x0_ref[:, :], x1_ref[:, :], x2_ref[:, :]
    d = ((xb - gxs[:, :]) ** 2 + (yb - gys[:, :]) ** 2) + (zb - gzs[:, :]) ** 2

    c0, c1, c2 = cw_ref[0, 0], cw_ref[0, 1], cw_ref[0, 2]
    c3, c4 = cw_ref[0, 3], cw_ref[0, 4]
    w0 = wv_ref[0, 0]

    def f(dd):
        rep = jnp.where(dd < 0.0, dd ** 2, 0.0)
        hyd = jnp.where(dd < 0.5, 1.0, jnp.where(dd < 1.5, 1.5 - dd, 0.0))
        hb = jnp.where(dd < -0.7, 1.0,
                       jnp.where(dd < 0.0, (1.0 / 0.7) * (0.0 - dd), 0.0))
        g1 = jnp.exp(-(dd / 0.5) ** 2)
        g2 = jnp.exp(-((dd - 3.0) / 2.0) ** 2)
        inter = c0 * rep + c1 * hyd + c2 * hb + c3 * g1 + c4 * g2
        return jnp.where(dd < 8.0, inter, 0.0) / (1.0 + w0 * 1.0)

    nv = jnp.sum(jnp.where(d < 192.0, 1.0, 0.0), axis=1, keepdims=True)
    padc = jnp.maximum(0.0, 32.0 - nv)
    xl, yl, zl = xl_ref[0, 0], xl_ref[0, 1], xl_ref[0, 2]
    dl = ((xb - xl) ** 2 + (yb - yl) ** 2) + (zb - zl) ** 2
    total = jnp.sum(padc * f(dl))

    keys = lax.bitcast_convert_type(d, jnp.int32)
    t = jnp.zeros((_BA, 1), jnp.int32)
    for b in range(30, -1, -1):
        cand = t | (1 << b)
        cnt = jnp.sum(jnp.where(keys < cand, 1.0, 0.0), axis=1, keepdims=True)
        t = jnp.where(cnt >= 32.0, t, cand)
    less = keys < t
    cntl = jnp.sum(jnp.where(less, 1.0, 0.0), axis=1, keepdims=True)
    t_f = lax.bitcast_convert_type(t, jnp.float32)
    total = total + jnp.sum(jnp.where(less, f(d), 0.0))
    total = total + jnp.sum((32.0 - cntl) * f(t_f))

    @pl.when(pid == 0)
    def _():
        out_ref[0, 0] = 0.0

    out_ref[0, 0] += total


def _run_k4(cids, gx, gy, gz, xc0, xc1, xc2, xlast, cw, wv):
    full = pl.BlockSpec((_NCELL, _ROW), lambda i, c: (0, 0))
    col = pl.BlockSpec((_BA, 1), lambda i, c: (i, 0))
    smem = pl.BlockSpec(memory_space=pltpu.SMEM)
    grid_spec = pltpu.PrefetchScalarGridSpec(
        num_scalar_prefetch=1,
        grid=(_N // _BA,),
        in_specs=[full, full, full, col, col, col, smem, smem, smem],
        out_specs=pl.BlockSpec(memory_space=pltpu.SMEM),
        scratch_shapes=[pltpu.VMEM((_BA, _ROW), jnp.float32)] * 3,
    )
    return pl.pallas_call(
        _k4_body,
        grid_spec=grid_spec,
        out_shape=jax.ShapeDtypeStruct((1, 1), jnp.float32),
    )(cids, gx, gy, gz, xc0, xc1, xc2, xlast, cw, wv)


def kernel(X, Z, combo_w, w):
    X = jnp.asarray(X, jnp.float32)
    x0, x1, x2 = X[:, 0], X[:, 1], X[:, 2]
    xc0, xc1, xc2 = x0[:, None], x1[:, None], x2[:, None]
    xr0, xr1, xr2 = x0[None, :], x1[None, :], x2[None, :]

    cids = _run_k0(xc0, xc1, xc2).reshape(_N)
    closest = _run_k1(xr0, xr1, xr2)
    src = jnp.asarray(_SRC_PAD)
    gxf, gyf, gzf = _run_k2(src, closest.reshape(-1), x0, x1, x2)
    out = _run_k4(
        cids,
        gxf.reshape(_NCELL, _ROW), gyf.reshape(_NCELL, _ROW),
        gzf.reshape(_NCELL, _ROW),
        xc0, xc1, xc2,
        X[_N - 1:_N, :],
        combo_w.reshape(1, 5).astype(jnp.float32),
        w.reshape(1, 1).astype(jnp.float32),
    )
    return out[0, 0]
